# baseline (speedup 1.0000x reference)
import functools

import jax
import jax.numpy as jnp
from jax import lax
from jax.experimental import pallas as pl
from jax.experimental.pallas import tpu as pltpu
from jax.experimental.pallas import tpu_sc as plsc

EMBED_DIM = 300
PAD_DIM = 384
BATCH = 4096
SEQ = 200
B_TOTAL = BATCH * SEQ
NUM_WORKERS = 32
B_PER_W = B_TOTAL // NUM_WORKERS
CHUNK = 128
HALF = 64
N_CHUNKS = B_PER_W // CHUNK
N_PAIRS = N_CHUNKS // 2
LANES = 16
D_UNROLL = 10

_mesh = plsc.VectorSubcoreMesh(core_axis_name="c", subcore_axis_name="s")


@functools.partial(
    pl.kernel,
    mesh=_mesh,
    out_type=jax.ShapeDtypeStruct((EMBED_DIM, SEQ, BATCH), jnp.float32),
    scratch_types=[
        pltpu.VMEM((2, HALF), jnp.int32),
        pltpu.VMEM((HALF, PAD_DIM), jnp.float32),
        pltpu.VMEM((HALF, PAD_DIM), jnp.float32),
        pltpu.VMEM((EMBED_DIM, CHUNK), jnp.float32),
        pltpu.VMEM((EMBED_DIM, CHUNK), jnp.float32),
        pltpu.SemaphoreType.DMA,
        pltpu.SemaphoreType.DMA,
        pltpu.SemaphoreType.DMA,
        pltpu.SemaphoreType.DMA,
    ],
    compiler_params=pltpu.CompilerParams(needs_layout_passes=False),
)
def _embed_t(idx_hbm, tab_hbm, out_hbm, idx_v, grow0, grow1, tb0, tb1,
             g0, g1, o0, o1):
    wid = lax.axis_index("s") * 2 + lax.axis_index("c")
    cbase = wid * N_CHUNKS
    grows = (grow0, grow1)
    tbufs = (tb0, tb1)
    gsem = (g0, g1)
    osem = (o0, o1)
    tabp = tab_hbm.at[:, pl.ds(0, PAD_DIM)]
    row_ids = tuple(lax.iota(jnp.int32, LANES) + g * LANES
                    for g in range(HALF // LANES))

    def out_slice(c):
        flat0 = (cbase + c) * CHUNK
        s = flat0 // BATCH
        b0 = flat0 % BATCH
        return out_hbm.at[pl.ds(0, EMBED_DIM), s, pl.ds(b0, CHUNK)]

    def start_gather(c, h, k):
        base = (cbase + c) * CHUNK + h * HALF
        pltpu.sync_copy(idx_hbm.at[pl.ds(base, HALF)], idx_v.at[k])
        pltpu.async_copy(tabp.at[idx_v.at[k]], grows[k], gsem[k])

    def wait_gather(k):
        pltpu.make_async_copy(tabp.at[idx_v.at[k]], grows[k],
                              gsem[k]).wait()

    def transpose(k, t, h):
        gref = grows[k]
        tref = tbufs[t]
        off = h * HALF

        def dgroup(dg, carry):
            for u in range(D_UNROLL):
                d = dg * D_UNROLL + u
                col = jnp.full((LANES,), 0, jnp.int32) + d
                for g in range(HALF // LANES):
                    tref[d, pl.ds(off + g * LANES, LANES)] = plsc.load_gather(
                        gref, [row_ids[g], col])
            return carry

        lax.fori_loop(0, EMBED_DIM // D_UNROLL, dgroup, 0)

    def start_store(c, t):
        pltpu.async_copy(tbufs[t], out_slice(c), osem[t])

    def wait_store(c, t):
        pltpu.make_async_copy(tbufs[t], out_slice(c), osem[t]).wait()

    start_gather(0, 0, 0)

    def body(p, carry):
        a = 2 * p
        b = a + 1
        start_gather(a, 1, 1)
        wait_gather(0)
        transpose(0, 0, 0)
        start_gather(b, 0, 0)
        wait_gather(1)
        transpose(1, 0, 1)

        @pl.when(p > 0)
        def _():
            wait_store(b - 2, 1)
        start_store(a, 0)
        start_gather(b, 1, 1)
        wait_gather(0)
        transpose(0, 1, 0)

        @pl.when(p < N_PAIRS - 1)
        def _():
            start_gather(a + 2, 0, 0)
        wait_gather(1)
        transpose(1, 1, 1)
        wait_store(a, 0)
        start_store(b, 1)
        return carry

    lax.fori_loop(0, N_PAIRS, body, 0)
    wait_store(N_CHUNKS - 1, 1)


def kernel(x, weight):
    idx = x.T.reshape(-1).astype(jnp.int32)
    out_t = _embed_t(idx, weight)
    return out_t.transpose(2, 1, 0)


# speedup vs baseline: 1.8157x; 1.8157x over previous
import functools

import jax
import jax.numpy as jnp
from jax import lax
from jax.experimental import pallas as pl
from jax.experimental.pallas import tpu as pltpu
from jax.experimental.pallas import tpu_sc as plsc

EMBED_DIM = 300
PAD_DIM = 384
BATCH = 4096
SEQ = 200
B_TOTAL = BATCH * SEQ
NUM_WORKERS = 32
B_PER_W = B_TOTAL // NUM_WORKERS
CHUNK = 128
HALF = 64
N_CHUNKS = B_PER_W // CHUNK
N_PAIRS = N_CHUNKS // 2
LANES = 16
D_UNROLL = 10

_mesh = plsc.VectorSubcoreMesh(core_axis_name="c", subcore_axis_name="s")


@functools.partial(
    pl.kernel,
    mesh=_mesh,
    out_type=jax.ShapeDtypeStruct((EMBED_DIM, SEQ, BATCH), jnp.float32),
    scratch_types=[
        pltpu.VMEM((2, HALF), jnp.int32),
        pltpu.VMEM((HALF, PAD_DIM), jnp.float32),
        pltpu.VMEM((HALF, PAD_DIM), jnp.float32),
        pltpu.VMEM((EMBED_DIM, CHUNK), jnp.float32),
        pltpu.VMEM((EMBED_DIM, CHUNK), jnp.float32),
        pltpu.SemaphoreType.DMA,
        pltpu.SemaphoreType.DMA,
        pltpu.SemaphoreType.DMA,
        pltpu.SemaphoreType.DMA,
    ],
    compiler_params=pltpu.CompilerParams(needs_layout_passes=False),
)
def _embed_t(idx_hbm, tab_hbm, out_hbm, idx_v, grow0, grow1, tb0, tb1,
             g0, g1, o0, o1):
    wid = lax.axis_index("s") * 2 + lax.axis_index("c")
    cbase = wid * N_CHUNKS
    grows = (grow0, grow1)
    tbufs = (tb0, tb1)
    gsem = (g0, g1)
    osem = (o0, o1)
    tabp = tab_hbm.at[:, pl.ds(0, PAD_DIM)]
    row_ids = tuple(lax.iota(jnp.int32, LANES) + g * LANES
                    for g in range(HALF // LANES))

    def out_slice(c):
        flat0 = (cbase + c) * CHUNK
        s = flat0 // BATCH
        b0 = flat0 % BATCH
        return out_hbm.at[pl.ds(0, EMBED_DIM), s, pl.ds(b0, CHUNK)]

    def start_gather(c, h, k):
        base = (cbase + c) * CHUNK + h * HALF
        pltpu.sync_copy(idx_hbm.at[pl.ds(base, HALF)], idx_v.at[k])
        pltpu.async_copy(tabp.at[idx_v.at[k]], grows[k], gsem[k])

    def wait_gather(k):
        pltpu.make_async_copy(tabp.at[idx_v.at[k]], grows[k],
                              gsem[k]).wait()

    def transpose(k, t, h):
        gref = grows[k]
        tref = tbufs[t]
        off = h * HALF

        @plsc.parallel_loop(0, EMBED_DIM, unroll=D_UNROLL)
        def dloop(d):
            col = jnp.full((LANES,), 0, jnp.int32) + d
            for g in range(HALF // LANES):
                tref[d, pl.ds(off + g * LANES, LANES)] = plsc.load_gather(
                    gref, [row_ids[g], col])

    def start_store(c, t):
        pltpu.async_copy(tbufs[t], out_slice(c), osem[t])

    def wait_store(c, t):
        pltpu.make_async_copy(tbufs[t], out_slice(c), osem[t]).wait()

    start_gather(0, 0, 0)

    def body(p, carry):
        a = 2 * p
        b = a + 1
        start_gather(a, 1, 1)
        wait_gather(0)
        transpose(0, 0, 0)
        start_gather(b, 0, 0)
        wait_gather(1)
        transpose(1, 0, 1)

        @pl.when(p > 0)
        def _():
            wait_store(b - 2, 1)
        start_store(a, 0)
        start_gather(b, 1, 1)
        wait_gather(0)
        transpose(0, 1, 0)

        @pl.when(p < N_PAIRS - 1)
        def _():
            start_gather(a + 2, 0, 0)
        wait_gather(1)
        transpose(1, 1, 1)
        wait_store(a, 0)
        start_store(b, 1)
        return carry

    lax.fori_loop(0, N_PAIRS, body, 0)
    wait_store(N_CHUNKS - 1, 1)


def kernel(x, weight):
    idx = x.T.reshape(-1).astype(jnp.int32)
    out_t = _embed_t(idx, weight)
    return out_t.transpose(2, 1, 0)


# speedup vs baseline: 3.3109x; 1.8235x over previous
import functools

import jax
import jax.numpy as jnp
from jax import lax
from jax.experimental import pallas as pl
from jax.experimental.pallas import tpu as pltpu
from jax.experimental.pallas import tpu_sc as plsc

EMBED_DIM = 300
PAD_DIM = 384
BATCH = 4096
SEQ = 200
B_TOTAL = BATCH * SEQ
NUM_WORKERS = 32
B_PER_W = B_TOTAL // NUM_WORKERS
CHUNK = 128
N_CHUNKS = B_PER_W // CHUNK
N_PAIRS = N_CHUNKS // 2

_mesh = plsc.VectorSubcoreMesh(core_axis_name="c", subcore_axis_name="s")


@functools.partial(
    pl.kernel,
    mesh=_mesh,
    out_type=jax.ShapeDtypeStruct((B_TOTAL, EMBED_DIM), jnp.float32),
    scratch_types=[
        pltpu.VMEM((2, CHUNK), jnp.int32),
        pltpu.VMEM((CHUNK, PAD_DIM), jnp.float32),
        pltpu.VMEM((CHUNK, PAD_DIM), jnp.float32),
        pltpu.SemaphoreType.DMA,
        pltpu.SemaphoreType.DMA,
        pltpu.SemaphoreType.DMA,
        pltpu.SemaphoreType.DMA,
    ],
)
def _embed_gather(idx_hbm, tab_hbm, out_hbm, idx_v, rows0, rows1, g0, g1,
                  o0, o1):
    wid = lax.axis_index("s") * 2 + lax.axis_index("c")
    wbase = wid * B_PER_W
    rows = (rows0, rows1)
    gsem = (g0, g1)
    osem = (o0, o1)
    tabp = tab_hbm.at[:, pl.ds(0, PAD_DIM)]

    def out_slice(g):
        return out_hbm.at[pl.ds(wbase + g * CHUNK, CHUNK), pl.ds(0, PAD_DIM)]

    def start_gather(g, k):
        pltpu.sync_copy(idx_hbm.at[pl.ds(wbase + g * CHUNK, CHUNK)],
                        idx_v.at[k])
        pltpu.async_copy(tabp.at[idx_v.at[k]], rows[k], gsem[k])

    def start_store(g, k):
        pltpu.make_async_copy(tabp.at[idx_v.at[k]], rows[k], gsem[k]).wait()
        pltpu.async_copy(rows[k], out_slice(g), osem[k])

    def wait_store(g, k):
        pltpu.make_async_copy(rows[k], out_slice(g), osem[k]).wait()

    start_gather(0, 0)

    def body(p, carry):
        a = 2 * p
        b = a + 1

        @pl.when(p > 0)
        def _():
            wait_store(b - 2, 1)

        start_gather(b, 1)
        start_store(a, 0)
        wait_store(a, 0)

        @pl.when(p < N_PAIRS - 1)
        def _():
            start_gather(a + 2, 0)

        start_store(b, 1)
        return carry

    lax.fori_loop(0, N_PAIRS, body, 0)
    wait_store(N_CHUNKS - 1, 1)


def kernel(x, weight):
    idx = x.reshape(-1).astype(jnp.int32)
    out = _embed_gather(idx, weight)
    return out.reshape(BATCH, SEQ, EMBED_DIM)
